# baseline (speedup 1.0000x reference)
import functools

import jax
import jax.numpy as jnp
import numpy as np
from jax import lax
from jax.experimental import pallas as pl
from jax.experimental.pallas import tpu as pltpu
from jax.experimental.pallas import tpu_sc as plsc

N = 10000
E = 320000
D = 128
H = 128
OUT = 64

NC = 2
NS = 16
NW = NC * NS
L = 16

EPT = E // NW
CH = 80
NCHUNK = EPT // CH
SBC = 25
NSB = NCHUNK // SBC
SBE = SBC * CH

NPAD = 10240
RPT = NPAD // NS

HQ = NPAD // 128
HB = 5000
HSTEPS = E // HB


def _tc_pre_body(x_ref, wp_ref, bp_ref, wms_ref, wmd_ref, bm_ref, dst_ref,
                 h_ref, a_ref, b_ref, cnt_ref):
    h = jnp.dot(x_ref[...], wp_ref[...],
                preferred_element_type=jnp.float32) + bp_ref[...]
    h_ref[...] = h
    a_ref[...] = jnp.dot(h, wms_ref[...], preferred_element_type=jnp.float32)
    b_ref[...] = (jnp.dot(h, wmd_ref[...],
                          preferred_element_type=jnp.float32) + bm_ref[...])

    iq = lax.broadcasted_iota(jnp.int32, (HQ, HB), 0)
    ir = lax.broadcasted_iota(jnp.int32, (HB, 128), 1)

    def _hstep(k, cnt):
        d = dst_ref[k, :]
        oq = (iq == (d // 128)[None, :]).astype(jnp.bfloat16)
        orr = (ir == (d % 128)[:, None]).astype(jnp.bfloat16)
        return cnt + jnp.dot(oq, orr, preferred_element_type=jnp.float32)

    cnt_ref[...] = lax.fori_loop(
        0, HSTEPS, _hstep, jnp.zeros((HQ, 128), jnp.float32))


def _tc_pre(x, w_proj, b_proj, wm_src, wm_dst, b_msg, dst):
    return pl.pallas_call(
        _tc_pre_body,
        out_shape=[
            jax.ShapeDtypeStruct((N, H), jnp.float32),
            jax.ShapeDtypeStruct((N, H), jnp.float32),
            jax.ShapeDtypeStruct((N, H), jnp.float32),
            jax.ShapeDtypeStruct((HQ, 128), jnp.float32),
        ],
    )(x, w_proj, b_proj.reshape(1, H), wm_src, wm_dst, b_msg.reshape(1, H),
      dst.reshape(HSTEPS, HB))


_sc_mesh = plsc.VectorSubcoreMesh(
    core_axis_name="c", subcore_axis_name="s", num_cores=NC, num_subcores=NS)


@functools.partial(
    pl.kernel,
    out_type=jax.ShapeDtypeStruct((NC, NPAD, H), jnp.float32),
    mesh=_sc_mesh,
    scratch_types=[
        pltpu.VMEM((SBE,), jnp.int32),
        pltpu.VMEM((SBE,), jnp.int32),
        pltpu.VMEM((CH,), jnp.int32),
        pltpu.VMEM((2, CH, H), jnp.float32),
        pltpu.VMEM((2, CH, H), jnp.float32),
        pltpu.VMEM_SHARED((NPAD, H), jnp.float32),
        pltpu.SemaphoreType.DMA((2,)),
        pltpu.SemaphoreType.DMA((2,)),
        pltpu.SemaphoreType.DMA,
    ],
)
def _sc_edge(a_hbm, b_hbm, src_hbm, dst_hbm, out_hbm,
             idx_s, idx_d, idx_w, a_buf, b_buf, acc, sga, sgb, sidx):
    c = lax.axis_index("c")
    s = lax.axis_index("s")
    wid = c * NS + s
    ebase = wid * EPT

    zeros = jnp.zeros((L,), jnp.float32)

    def _zero_row(r, _):
        for j in range(H // L):
            a_buf[0, r, pl.ds(j * L, L)] = zeros
        return 0
    lax.fori_loop(0, CH, _zero_row, 0)
    rbase = s * RPT
    for k in range(RPT // CH):
        pltpu.sync_copy(a_buf.at[0], acc.at[pl.ds(rbase + k * CH, CH), :])

    plsc.subcore_barrier()

    def _issue_g(slot, ci):
        pltpu.async_copy(a_hbm.at[idx_s.at[pl.ds(ci * CH, CH)]],
                         a_buf.at[slot], sga.at[slot])
        pltpu.async_copy(b_hbm.at[idx_d.at[pl.ds(ci * CH, CH)]],
                         b_buf.at[slot], sgb.at[slot])

    def _wait_g(slot, ci):
        pltpu.make_async_copy(a_hbm.at[idx_s.at[pl.ds(ci * CH, CH)]],
                              a_buf.at[slot], sga.at[slot]).wait()
        pltpu.make_async_copy(b_hbm.at[idx_d.at[pl.ds(ci * CH, CH)]],
                              b_buf.at[slot], sgb.at[slot]).wait()

    def _sblock(sb, _):
        base = ebase + sb * SBE
        pltpu.sync_copy(src_hbm.at[pl.ds(base, SBE)], idx_s)
        pltpu.sync_copy(dst_hbm.at[pl.ds(base, SBE)], idx_d)
        _issue_g(0, 0)

        def _chunk(ci, _):
            p = lax.rem(ci, 2)
            q = 1 - p

            @pl.when(ci + 1 < SBC)
            def _():
                _issue_g(q, ci + 1)

            _wait_g(p, ci)

            for j in range(CH // L):
                idx_w[pl.ds(j * L, L)] = idx_d[pl.ds(ci * CH + j * L, L)]

            def _row(r, _):
                for j in range(H // L):
                    sl = pl.ds(j * L, L)
                    a_buf[p, r, sl] = jnp.maximum(
                        a_buf[p, r, sl] + b_buf[p, r, sl], 0.0)
                return 0
            lax.fori_loop(0, CH, _row, 0)

            pltpu.sync_copy(a_buf.at[p], acc.at[idx_w], add=True)
            return 0
        lax.fori_loop(0, SBC, _chunk, 0)
        return 0
    lax.fori_loop(0, NSB, _sblock, 0)

    plsc.subcore_barrier()

    for k in range(RPT // CH):
        off = rbase + k * CH
        pltpu.sync_copy(acc.at[pl.ds(off, CH), :],
                        out_hbm.at[c, pl.ds(off, CH), :])


def _tc_post_body(h_ref, p_ref, cnt_ref, wnt_ref, wnb_ref, bn_ref, g_ref,
                  be_ref, wo_ref, bo_ref, o_ref):
    h = h_ref[...]
    psum = p_ref[0] + p_ref[1]
    iq = lax.broadcasted_iota(jnp.int32, (NPAD, HQ), 0)
    qq = lax.broadcasted_iota(jnp.int32, (NPAD, HQ), 1)
    oq = (iq // 128 == qq).astype(jnp.float32)
    ir = lax.broadcasted_iota(jnp.int32, (NPAD, 128), 0)
    rr = lax.broadcasted_iota(jnp.int32, (NPAD, 128), 1)
    orr = (ir % 128 == rr).astype(jnp.float32)
    tmp = jnp.dot(oq, cnt_ref[...], preferred_element_type=jnp.float32)
    cnt = jnp.sum(tmp * orr, axis=1, keepdims=True)
    pooled = (psum / jnp.maximum(cnt, 1.0))[:N, :]
    nxt = jnp.dot(h, wnt_ref[...], preferred_element_type=jnp.float32)
    nxt = nxt + jnp.dot(pooled, wnb_ref[...],
                        preferred_element_type=jnp.float32)
    nxt = jnp.maximum(nxt + bn_ref[...], 0.0)
    mu = jnp.mean(nxt, axis=-1, keepdims=True)
    var = jnp.mean((nxt - mu) * (nxt - mu), axis=-1, keepdims=True)
    ln = g_ref[...] * (nxt - mu) / jnp.sqrt(var + 1e-5) + be_ref[...]
    new_h = h + ln
    agg = jnp.sum(new_h, axis=0, keepdims=True) * (1.0 / N)
    o_ref[...] = jnp.dot(agg, wo_ref[...],
                         preferred_element_type=jnp.float32) + bo_ref[...]


def _tc_post(h, partials, counts, wn_top, wn_bot, b_next, gamma, beta,
             w_out, b_out):
    return pl.pallas_call(
        _tc_post_body,
        out_shape=jax.ShapeDtypeStruct((1, OUT), jnp.float32),
    )(h, partials, counts, wn_top, wn_bot, b_next.reshape(1, H),
      gamma.reshape(1, H), beta.reshape(1, H), w_out, b_out.reshape(1, OUT))


_PERM = np.empty((H,), dtype=np.int32)
for _g in range(H // 32):
    for _k in range(16):
        _PERM[32 * _g + 2 * _k] = 32 * _g + _k
        _PERM[32 * _g + 2 * _k + 1] = 32 * _g + 16 + _k


@jax.jit
def kernel(x, edge_index, W_proj, b_proj, W_msg, b_msg, W_next, b_next,
           gamma, beta, W_out, b_out):
    src = edge_index[0]
    dst = edge_index[1]
    h, a, b, counts = _tc_pre(x, W_proj, b_proj, W_msg[:H], W_msg[H:],
                              b_msg, dst)
    partials = _sc_edge(a, b, src, dst)
    return _tc_post(h, partials, counts, W_next[:H], W_next[H:], b_next,
                    gamma, beta, W_out, b_out)


# speedup vs baseline: 2.7207x; 2.7207x over previous
import functools

import jax
import jax.numpy as jnp
import numpy as np
from jax import lax
from jax.experimental import pallas as pl
from jax.experimental.pallas import tpu as pltpu
from jax.experimental.pallas import tpu_sc as plsc

N = 10000
E = 320000
D = 128
H = 128
OUT = 64

NC = 2
NS = 16
NW = NC * NS
L = 16

EPT = E // NW
CH = 80
NCHUNK = EPT // CH
SBC = 25
NSB = NCHUNK // SBC
SBE = SBC * CH

NPAD = 10240
RPT = NPAD // NS

HQ = NPAD // 128
HB = 5000
HSTEPS = E // HB


def _tc_pre_body(x_ref, wp_ref, bp_ref, wms_ref, wmd_ref, bm_ref,
                 h_ref, a_ref, b_ref):
    h = jnp.dot(x_ref[...], wp_ref[...],
                preferred_element_type=jnp.float32) + bp_ref[...]
    h_ref[...] = h
    a_ref[...] = jnp.dot(h, wms_ref[...], preferred_element_type=jnp.float32)
    b_ref[...] = (jnp.dot(h, wmd_ref[...],
                          preferred_element_type=jnp.float32) + bm_ref[...])


def _tc_pre(x, w_proj, b_proj, wm_src, wm_dst, b_msg):
    return pl.pallas_call(
        _tc_pre_body,
        out_shape=[
            jax.ShapeDtypeStruct((N, H), jnp.float32),
            jax.ShapeDtypeStruct((N, H), jnp.float32),
            jax.ShapeDtypeStruct((N, H), jnp.float32),
        ],
    )(x, w_proj, b_proj.reshape(1, H), wm_src, wm_dst, b_msg.reshape(1, H))


def _tc_hist_body(dst_ref, cnt_ref):
    iq = lax.broadcasted_iota(jnp.int32, (HQ, HB), 0)
    ir = lax.broadcasted_iota(jnp.int32, (HB, 128), 1)

    def _hstep(k, cnt):
        d = dst_ref[k, :]
        oq = (iq == (d // 128)[None, :]).astype(jnp.bfloat16)
        orr = (ir == (d % 128)[:, None]).astype(jnp.bfloat16)
        return cnt + jnp.dot(oq, orr, preferred_element_type=jnp.float32)

    cnt_ref[...] = lax.fori_loop(
        0, HSTEPS, _hstep, jnp.zeros((HQ, 128), jnp.float32))


def _tc_hist(dst):
    return pl.pallas_call(
        _tc_hist_body,
        out_shape=jax.ShapeDtypeStruct((HQ, 128), jnp.float32),
    )(dst.reshape(HSTEPS, HB))


_sc_mesh = plsc.VectorSubcoreMesh(
    core_axis_name="c", subcore_axis_name="s", num_cores=NC, num_subcores=NS)


@functools.partial(
    pl.kernel,
    out_type=jax.ShapeDtypeStruct((NC, NPAD, H), jnp.float32),
    mesh=_sc_mesh,
    scratch_types=[
        pltpu.VMEM((SBE,), jnp.int32),
        pltpu.VMEM((SBE,), jnp.int32),
        pltpu.VMEM((CH,), jnp.int32),
        pltpu.VMEM((CH, H), jnp.float32),
        pltpu.VMEM((CH, H), jnp.float32),
        pltpu.VMEM((CH, H), jnp.float32),
        pltpu.VMEM((CH, H), jnp.float32),
        pltpu.VMEM_SHARED((NPAD, H), jnp.float32),
        pltpu.SemaphoreType.DMA,
        pltpu.SemaphoreType.DMA,
        pltpu.SemaphoreType.DMA,
        pltpu.SemaphoreType.DMA,
    ],
)
def _sc_edge(a_hbm, b_hbm, src_hbm, dst_hbm, out_hbm,
             idx_s, idx_d, idx_w, a0, b0, a1, b1, acc,
             sa0, sb0, sa1, sb1):
    c = lax.axis_index("c")
    s = lax.axis_index("s")
    wid = c * NS + s
    ebase = wid * EPT

    zeros = jnp.zeros((L,), jnp.float32)

    def _zero_row(r, _):
        for j in range(H // L):
            a0[r, pl.ds(j * L, L)] = zeros
        return 0
    lax.fori_loop(0, CH, _zero_row, 0)
    rbase = s * RPT
    for k in range(RPT // CH):
        pltpu.sync_copy(a0, acc.at[pl.ds(rbase + k * CH, CH), :])

    plsc.subcore_barrier()

    def _issue_g(abuf, bbuf, sa, sb, ci):
        pltpu.async_copy(a_hbm.at[idx_s.at[pl.ds(ci * CH, CH)]], abuf, sa)
        pltpu.async_copy(b_hbm.at[idx_d.at[pl.ds(ci * CH, CH)]], bbuf, sb)

    def _wait_g(abuf, bbuf, sa, sb, ci):
        pltpu.make_async_copy(a_hbm.at[idx_s.at[pl.ds(ci * CH, CH)]],
                              abuf, sa).wait()
        pltpu.make_async_copy(b_hbm.at[idx_d.at[pl.ds(ci * CH, CH)]],
                              bbuf, sb).wait()

    def _reduce(abuf, bbuf, ci):
        for j in range(CH // L):
            idx_w[pl.ds(j * L, L)] = idx_d[pl.ds(ci * CH + j * L, L)]

        def _row(r, _):
            for j in range(H // L):
                sl = pl.ds(j * L, L)
                abuf[r, sl] = jnp.maximum(abuf[r, sl] + bbuf[r, sl], 0.0)
            return 0
        lax.fori_loop(0, CH, _row, 0)
        pltpu.sync_copy(abuf, acc.at[idx_w], add=True)

    def _sblock(sb, _):
        base = ebase + sb * SBE
        pltpu.sync_copy(src_hbm.at[pl.ds(base, SBE)], idx_s)
        pltpu.sync_copy(dst_hbm.at[pl.ds(base, SBE)], idx_d)
        _issue_g(a0, b0, sa0, sb0, 0)

        def _pair(t, _):
            ci0 = 2 * t
            ci1 = ci0 + 1
            _issue_g(a1, b1, sa1, sb1, ci1)
            _wait_g(a0, b0, sa0, sb0, ci0)
            _reduce(a0, b0, ci0)
            _issue_g(a0, b0, sa0, sb0, ci0 + 2)
            _wait_g(a1, b1, sa1, sb1, ci1)
            _reduce(a1, b1, ci1)
            return 0
        lax.fori_loop(0, SBC // 2, _pair, 0)
        _wait_g(a0, b0, sa0, sb0, SBC - 1)
        _reduce(a0, b0, SBC - 1)
        return 0
    lax.fori_loop(0, NSB, _sblock, 0)

    plsc.subcore_barrier()

    for k in range(RPT // CH):
        off = rbase + k * CH
        pltpu.sync_copy(acc.at[pl.ds(off, CH), :],
                        out_hbm.at[c, pl.ds(off, CH), :])


def _tc_post_body(h_ref, p_ref, cnt_ref, wnt_ref, wnb_ref, bn_ref, g_ref,
                  be_ref, wo_ref, bo_ref, o_ref):
    h = h_ref[...]
    psum = p_ref[0] + p_ref[1]
    iq = lax.broadcasted_iota(jnp.int32, (NPAD, HQ), 0)
    qq = lax.broadcasted_iota(jnp.int32, (NPAD, HQ), 1)
    oq = (iq // 128 == qq).astype(jnp.float32)
    ir = lax.broadcasted_iota(jnp.int32, (NPAD, 128), 0)
    rr = lax.broadcasted_iota(jnp.int32, (NPAD, 128), 1)
    orr = (ir % 128 == rr).astype(jnp.float32)
    tmp = jnp.dot(oq, cnt_ref[...], preferred_element_type=jnp.float32)
    cnt = jnp.sum(tmp * orr, axis=1, keepdims=True)
    pooled = (psum / jnp.maximum(cnt, 1.0))[:N, :]
    nxt = jnp.dot(h, wnt_ref[...], preferred_element_type=jnp.float32)
    nxt = nxt + jnp.dot(pooled, wnb_ref[...],
                        preferred_element_type=jnp.float32)
    nxt = jnp.maximum(nxt + bn_ref[...], 0.0)
    mu = jnp.mean(nxt, axis=-1, keepdims=True)
    var = jnp.mean((nxt - mu) * (nxt - mu), axis=-1, keepdims=True)
    ln = g_ref[...] * (nxt - mu) / jnp.sqrt(var + 1e-5) + be_ref[...]
    new_h = h + ln
    agg = jnp.sum(new_h, axis=0, keepdims=True) * (1.0 / N)
    o_ref[...] = jnp.dot(agg, wo_ref[...],
                         preferred_element_type=jnp.float32) + bo_ref[...]


def _tc_post(h, partials, counts, wn_top, wn_bot, b_next, gamma, beta,
             w_out, b_out):
    return pl.pallas_call(
        _tc_post_body,
        out_shape=jax.ShapeDtypeStruct((1, OUT), jnp.float32),
    )(h, partials, counts, wn_top, wn_bot, b_next.reshape(1, H),
      gamma.reshape(1, H), beta.reshape(1, H), w_out, b_out.reshape(1, OUT))


_PERM = np.empty((H,), dtype=np.int32)
for _g in range(H // 32):
    for _k in range(16):
        _PERM[32 * _g + 2 * _k] = 32 * _g + _k
        _PERM[32 * _g + 2 * _k + 1] = 32 * _g + 16 + _k


@jax.jit
def kernel(x, edge_index, W_proj, b_proj, W_msg, b_msg, W_next, b_next,
           gamma, beta, W_out, b_out):
    src = edge_index[0]
    dst = edge_index[1]
    h, a, b = _tc_pre(x, W_proj, b_proj, W_msg[:H], W_msg[H:], b_msg)
    partials = _sc_edge(a, b, src, dst)
    counts = _tc_hist(dst)
    return _tc_post(h, partials, counts, W_next[:H], W_next[H:], b_next,
                    gamma, beta, W_out, b_out)


# speedup vs baseline: 2.7270x; 1.0023x over previous
import functools

import jax
import jax.numpy as jnp
import numpy as np
from jax import lax
from jax.experimental import pallas as pl
from jax.experimental.pallas import tpu as pltpu
from jax.experimental.pallas import tpu_sc as plsc

N = 10000
E = 320000
D = 128
H = 128
OUT = 64

NC = 2
NS = 16
NW = NC * NS
L = 16

EPT = E // NW
CH = 80
NCHUNK = EPT // CH
SBC = 25
NSB = NCHUNK // SBC
SBE = SBC * CH

NPAD = 10240
RPT = NPAD // NS

HQ = NPAD // 128
HB = 5000
HSTEPS = E // HB


def _tc_pre_body(x_ref, wp_ref, bp_ref, wm_ref, bm_ref,
                 h_ref, a_ref, b_ref):
    h = jnp.dot(x_ref[...], wp_ref[...],
                preferred_element_type=jnp.float32) + bp_ref[...]
    h_ref[...] = h
    a_ref[...] = jnp.dot(h, wm_ref[:H, :],
                         preferred_element_type=jnp.float32)
    b_ref[...] = (jnp.dot(h, wm_ref[H:, :],
                          preferred_element_type=jnp.float32) + bm_ref[...])


def _tc_pre(x, w_proj, b_proj, w_msg, b_msg):
    return pl.pallas_call(
        _tc_pre_body,
        out_shape=[
            jax.ShapeDtypeStruct((N, H), jnp.float32),
            jax.ShapeDtypeStruct((N, H), jnp.float32),
            jax.ShapeDtypeStruct((N, H), jnp.float32),
        ],
    )(x, w_proj, b_proj.reshape(1, H), w_msg, b_msg.reshape(1, H))


def _tc_hist_body(dst_ref, cnt_ref):
    iq = lax.broadcasted_iota(jnp.int32, (HQ, HB), 0)
    ir = lax.broadcasted_iota(jnp.int32, (HB, 128), 1)

    def _hstep(k, cnt):
        d = dst_ref[k, :]
        oq = (iq == (d // 128)[None, :]).astype(jnp.bfloat16)
        orr = (ir == (d % 128)[:, None]).astype(jnp.bfloat16)
        return cnt + jnp.dot(oq, orr, preferred_element_type=jnp.float32)

    cnt_ref[...] = lax.fori_loop(
        0, HSTEPS, _hstep, jnp.zeros((HQ, 128), jnp.float32))


def _tc_hist(dst):
    return pl.pallas_call(
        _tc_hist_body,
        out_shape=jax.ShapeDtypeStruct((HQ, 128), jnp.float32),
    )(dst.reshape(HSTEPS, HB))


_sc_mesh = plsc.VectorSubcoreMesh(
    core_axis_name="c", subcore_axis_name="s", num_cores=NC, num_subcores=NS)


@functools.partial(
    pl.kernel,
    out_type=jax.ShapeDtypeStruct((NC, NPAD, H), jnp.float32),
    mesh=_sc_mesh,
    scratch_types=[
        pltpu.VMEM((SBE,), jnp.int32),
        pltpu.VMEM((SBE,), jnp.int32),
        pltpu.VMEM((CH,), jnp.int32),
        pltpu.VMEM((CH, H), jnp.float32),
        pltpu.VMEM((CH, H), jnp.float32),
        pltpu.VMEM((CH, H), jnp.float32),
        pltpu.VMEM((CH, H), jnp.float32),
        pltpu.VMEM_SHARED((NPAD, H), jnp.float32),
        pltpu.SemaphoreType.DMA,
        pltpu.SemaphoreType.DMA,
        pltpu.SemaphoreType.DMA,
        pltpu.SemaphoreType.DMA,
    ],
)
def _sc_edge(a_hbm, b_hbm, src_hbm, dst_hbm, out_hbm,
             idx_s, idx_d, idx_w, a0, b0, a1, b1, acc,
             sa0, sb0, sa1, sb1):
    c = lax.axis_index("c")
    s = lax.axis_index("s")
    wid = c * NS + s
    ebase = wid * EPT

    zeros = jnp.zeros((L,), jnp.float32)

    def _zero_row(r, _):
        for j in range(H // L):
            a0[r, pl.ds(j * L, L)] = zeros
        return 0
    lax.fori_loop(0, CH, _zero_row, 0)
    rbase = s * RPT
    for k in range(RPT // CH):
        pltpu.sync_copy(a0, acc.at[pl.ds(rbase + k * CH, CH), :])

    plsc.subcore_barrier()

    def _issue_g(abuf, bbuf, sa, sb, ci):
        pltpu.async_copy(a_hbm.at[idx_s.at[pl.ds(ci * CH, CH)]], abuf, sa)
        pltpu.async_copy(b_hbm.at[idx_d.at[pl.ds(ci * CH, CH)]], bbuf, sb)

    def _wait_g(abuf, bbuf, sa, sb, ci):
        pltpu.make_async_copy(a_hbm.at[idx_s.at[pl.ds(ci * CH, CH)]],
                              abuf, sa).wait()
        pltpu.make_async_copy(b_hbm.at[idx_d.at[pl.ds(ci * CH, CH)]],
                              bbuf, sb).wait()

    def _reduce(abuf, bbuf, ci):
        for j in range(CH // L):
            idx_w[pl.ds(j * L, L)] = idx_d[pl.ds(ci * CH + j * L, L)]

        def _row(r, _):
            for j in range(H // L):
                sl = pl.ds(j * L, L)
                abuf[r, sl] = jnp.maximum(abuf[r, sl] + bbuf[r, sl], 0.0)
            return 0
        lax.fori_loop(0, CH, _row, 0)
        pltpu.sync_copy(abuf, acc.at[idx_w], add=True)

    def _sblock(sb, _):
        base = ebase + sb * SBE
        pltpu.sync_copy(src_hbm.at[pl.ds(base, SBE)], idx_s)
        pltpu.sync_copy(dst_hbm.at[pl.ds(base, SBE)], idx_d)
        _issue_g(a0, b0, sa0, sb0, 0)

        def _pair(t, _):
            ci0 = 2 * t
            ci1 = ci0 + 1
            _issue_g(a1, b1, sa1, sb1, ci1)
            _wait_g(a0, b0, sa0, sb0, ci0)
            _reduce(a0, b0, ci0)
            _issue_g(a0, b0, sa0, sb0, ci0 + 2)
            _wait_g(a1, b1, sa1, sb1, ci1)
            _reduce(a1, b1, ci1)
            return 0
        lax.fori_loop(0, SBC // 2, _pair, 0)
        _wait_g(a0, b0, sa0, sb0, SBC - 1)
        _reduce(a0, b0, SBC - 1)
        return 0
    lax.fori_loop(0, NSB, _sblock, 0)

    plsc.subcore_barrier()

    for k in range(RPT // CH):
        off = rbase + k * CH
        pltpu.sync_copy(acc.at[pl.ds(off, CH), :],
                        out_hbm.at[c, pl.ds(off, CH), :])


def _tc_post_body(h_ref, p_ref, cnt_ref, wn_ref, bn_ref, g_ref,
                  be_ref, wo_ref, bo_ref, o_ref):
    h = h_ref[...]
    psum = p_ref[0] + p_ref[1]
    iq = lax.broadcasted_iota(jnp.int32, (NPAD, HQ), 0)
    qq = lax.broadcasted_iota(jnp.int32, (NPAD, HQ), 1)
    oq = (iq // 128 == qq).astype(jnp.float32)
    ir = lax.broadcasted_iota(jnp.int32, (NPAD, 128), 0)
    rr = lax.broadcasted_iota(jnp.int32, (NPAD, 128), 1)
    orr = (ir % 128 == rr).astype(jnp.float32)
    tmp = jnp.dot(oq, cnt_ref[...], preferred_element_type=jnp.float32)
    cnt = jnp.sum(tmp * orr, axis=1, keepdims=True)
    pooled = (psum / jnp.maximum(cnt, 1.0))[:N, :]
    nxt = jnp.dot(h, wn_ref[:H, :], preferred_element_type=jnp.float32)
    nxt = nxt + jnp.dot(pooled, wn_ref[H:, :],
                        preferred_element_type=jnp.float32)
    nxt = jnp.maximum(nxt + bn_ref[...], 0.0)
    mu = jnp.mean(nxt, axis=-1, keepdims=True)
    var = jnp.mean((nxt - mu) * (nxt - mu), axis=-1, keepdims=True)
    ln = g_ref[...] * (nxt - mu) / jnp.sqrt(var + 1e-5) + be_ref[...]
    new_h = h + ln
    agg = jnp.sum(new_h, axis=0, keepdims=True) * (1.0 / N)
    o_ref[...] = jnp.dot(agg, wo_ref[...],
                         preferred_element_type=jnp.float32) + bo_ref[...]


def _tc_post(h, partials, counts, w_next, b_next, gamma, beta,
             w_out, b_out):
    return pl.pallas_call(
        _tc_post_body,
        out_shape=jax.ShapeDtypeStruct((1, OUT), jnp.float32),
    )(h, partials, counts, w_next, b_next.reshape(1, H),
      gamma.reshape(1, H), beta.reshape(1, H), w_out, b_out.reshape(1, OUT))


_PERM = np.empty((H,), dtype=np.int32)
for _g in range(H // 32):
    for _k in range(16):
        _PERM[32 * _g + 2 * _k] = 32 * _g + _k
        _PERM[32 * _g + 2 * _k + 1] = 32 * _g + 16 + _k


@jax.jit
def kernel(x, edge_index, W_proj, b_proj, W_msg, b_msg, W_next, b_next,
           gamma, beta, W_out, b_out):
    h, a, b = _tc_pre(x, W_proj, b_proj, W_msg, b_msg)
    src = edge_index[0]
    dst = edge_index[1]
    partials = _sc_edge(a, b, src, dst)
    counts = _tc_hist(dst)
    return _tc_post(h, partials, counts, W_next, b_next,
                    gamma, beta, W_out, b_out)


# speedup vs baseline: 2.7300x; 1.0011x over previous
import functools

import jax
import jax.numpy as jnp
from jax import lax
from jax.experimental import pallas as pl
from jax.experimental.pallas import tpu as pltpu
from jax.experimental.pallas import tpu_sc as plsc

N = 10000
E = 320000
D = 128
H = 128
OUT = 64

NC = 2
NS = 16
NW = NC * NS
L = 16

EPT = E // NW
CH = 80
NCHUNK = EPT // CH
SBC = 25
NSB = NCHUNK // SBC
SBE = SBC * CH

NPAD = 10240
RPT = NPAD // NS

HQ = NPAD // 128
HB = 5000
HSTEPS = E // HB


def _tc_pre_body(x_ref, wp_ref, bp_ref, wm_ref, bm_ref,
                 h_ref, a_ref, b_ref):
    h = jnp.dot(x_ref[...], wp_ref[...],
                preferred_element_type=jnp.float32) + bp_ref[...]
    h_ref[...] = h
    a_ref[...] = jnp.dot(h, wm_ref[:H, :],
                         preferred_element_type=jnp.float32)
    b_ref[...] = (jnp.dot(h, wm_ref[H:, :],
                          preferred_element_type=jnp.float32) + bm_ref[...])


def _tc_pre(x, w_proj, b_proj, w_msg, b_msg):
    return pl.pallas_call(
        _tc_pre_body,
        out_shape=[
            jax.ShapeDtypeStruct((N, H), jnp.float32),
            jax.ShapeDtypeStruct((N, H), jnp.float32),
            jax.ShapeDtypeStruct((N, H), jnp.float32),
        ],
    )(x, w_proj, b_proj.reshape(1, H), w_msg, b_msg.reshape(1, H))


def _tc_hist_body(dst_ref, cnt_ref):
    iq = lax.broadcasted_iota(jnp.int32, (HQ, HB), 0)
    ir = lax.broadcasted_iota(jnp.int32, (HB, 128), 1)

    def _hstep(k, cnt):
        d = dst_ref[k, :]
        oq = (iq == (d // 128)[None, :]).astype(jnp.bfloat16)
        orr = (ir == (d % 128)[:, None]).astype(jnp.bfloat16)
        return cnt + jnp.dot(oq, orr, preferred_element_type=jnp.float32)

    cnt_ref[...] = lax.fori_loop(
        0, HSTEPS, _hstep, jnp.zeros((HQ, 128), jnp.float32))


def _tc_hist(dst):
    return pl.pallas_call(
        _tc_hist_body,
        out_shape=jax.ShapeDtypeStruct((HQ, 128), jnp.float32),
    )(dst.reshape(HSTEPS, HB))


_sc_mesh = plsc.VectorSubcoreMesh(
    core_axis_name="c", subcore_axis_name="s", num_cores=NC, num_subcores=NS)


@functools.partial(
    pl.kernel,
    out_type=jax.ShapeDtypeStruct((NC, NPAD, H), jnp.float32),
    mesh=_sc_mesh,
    scratch_types=[
        pltpu.VMEM((SBE,), jnp.int32),
        pltpu.VMEM((SBE,), jnp.int32),
        pltpu.VMEM((CH,), jnp.int32),
        pltpu.VMEM((CH, H), jnp.float32),
        pltpu.VMEM((CH, H), jnp.float32),
        pltpu.VMEM((CH, H), jnp.float32),
        pltpu.VMEM((CH, H), jnp.float32),
        pltpu.VMEM_SHARED((NPAD, H), jnp.float32),
        pltpu.SemaphoreType.DMA,
        pltpu.SemaphoreType.DMA,
        pltpu.SemaphoreType.DMA,
        pltpu.SemaphoreType.DMA,
    ],
)
def _sc_edge(a_hbm, b_hbm, src_hbm, dst_hbm, out_hbm,
             idx_s, idx_d, idx_w, a0, b0, a1, b1, acc,
             sa0, sb0, sa1, sb1):
    c = lax.axis_index("c")
    s = lax.axis_index("s")
    wid = c * NS + s
    ebase = wid * EPT

    zeros = jnp.zeros((L,), jnp.float32)

    def _zero_row(r, _):
        for j in range(H // L):
            a0[r, pl.ds(j * L, L)] = zeros
        return 0
    lax.fori_loop(0, CH, _zero_row, 0)
    rbase = s * RPT
    for k in range(RPT // CH):
        pltpu.sync_copy(a0, acc.at[pl.ds(rbase + k * CH, CH), :])

    plsc.subcore_barrier()

    def _issue_g(abuf, bbuf, sa, sb, ci):
        pltpu.async_copy(a_hbm.at[idx_s.at[pl.ds(ci * CH, CH)]], abuf, sa)
        pltpu.async_copy(b_hbm.at[idx_d.at[pl.ds(ci * CH, CH)]], bbuf, sb)

    def _wait_g(abuf, bbuf, sa, sb, ci):
        pltpu.make_async_copy(a_hbm.at[idx_s.at[pl.ds(ci * CH, CH)]],
                              abuf, sa).wait()
        pltpu.make_async_copy(b_hbm.at[idx_d.at[pl.ds(ci * CH, CH)]],
                              bbuf, sb).wait()

    def _reduce(abuf, bbuf, ci):
        for j in range(CH // L):
            idx_w[pl.ds(j * L, L)] = idx_d[pl.ds(ci * CH + j * L, L)]

        def _row(r, _):
            for j in range(H // L):
                sl = pl.ds(j * L, L)
                abuf[r, sl] = jnp.maximum(abuf[r, sl] + bbuf[r, sl], 0.0)
            return 0
        lax.fori_loop(0, CH, _row, 0)
        pltpu.sync_copy(abuf, acc.at[idx_w], add=True)

    def _sblock(sb, _):
        base = ebase + sb * SBE
        pltpu.sync_copy(src_hbm.at[pl.ds(base, SBE)], idx_s)
        pltpu.sync_copy(dst_hbm.at[pl.ds(base, SBE)], idx_d)
        _issue_g(a0, b0, sa0, sb0, 0)

        def _pair(t, _):
            ci0 = 2 * t
            ci1 = ci0 + 1
            _issue_g(a1, b1, sa1, sb1, ci1)
            _wait_g(a0, b0, sa0, sb0, ci0)
            _reduce(a0, b0, ci0)
            _issue_g(a0, b0, sa0, sb0, ci0 + 2)
            _wait_g(a1, b1, sa1, sb1, ci1)
            _reduce(a1, b1, ci1)
            return 0
        lax.fori_loop(0, SBC // 2, _pair, 0)
        _wait_g(a0, b0, sa0, sb0, SBC - 1)
        _reduce(a0, b0, SBC - 1)
        return 0
    lax.fori_loop(0, NSB, _sblock, 0)

    plsc.subcore_barrier()

    for k in range(RPT // CH):
        off = rbase + k * CH
        pltpu.sync_copy(acc.at[pl.ds(off, CH), :],
                        out_hbm.at[c, pl.ds(off, CH), :])


def _tc_post_body(h_ref, p_ref, cnt_ref, wn_ref, bn_ref, g_ref,
                  be_ref, wo_ref, bo_ref, o_ref):
    h = h_ref[...]
    psum = p_ref[0] + p_ref[1]
    iq = lax.broadcasted_iota(jnp.int32, (NPAD, HQ), 0)
    qq = lax.broadcasted_iota(jnp.int32, (NPAD, HQ), 1)
    oq = (iq // 128 == qq).astype(jnp.float32)
    ir = lax.broadcasted_iota(jnp.int32, (NPAD, 128), 0)
    rr = lax.broadcasted_iota(jnp.int32, (NPAD, 128), 1)
    orr = (ir % 128 == rr).astype(jnp.float32)
    tmp = jnp.dot(oq, cnt_ref[...], preferred_element_type=jnp.float32)
    cnt = jnp.sum(tmp * orr, axis=1, keepdims=True)
    pooled = (psum / jnp.maximum(cnt, 1.0))[:N, :]
    nxt = jnp.dot(h, wn_ref[:H, :], preferred_element_type=jnp.float32)
    nxt = nxt + jnp.dot(pooled, wn_ref[H:, :],
                        preferred_element_type=jnp.float32)
    nxt = jnp.maximum(nxt + bn_ref[...], 0.0)
    mu = jnp.mean(nxt, axis=-1, keepdims=True)
    var = jnp.mean((nxt - mu) * (nxt - mu), axis=-1, keepdims=True)
    ln = g_ref[...] * (nxt - mu) / jnp.sqrt(var + 1e-5) + be_ref[...]
    new_h = h + ln
    agg = jnp.sum(new_h, axis=0, keepdims=True) * (1.0 / N)
    o_ref[...] = jnp.dot(agg, wo_ref[...],
                         preferred_element_type=jnp.float32) + bo_ref[...]


def _tc_post(h, partials, counts, w_next, b_next, gamma, beta,
             w_out, b_out):
    return pl.pallas_call(
        _tc_post_body,
        out_shape=jax.ShapeDtypeStruct((1, OUT), jnp.float32),
    )(h, partials, counts, w_next, b_next.reshape(1, H),
      gamma.reshape(1, H), beta.reshape(1, H), w_out, b_out.reshape(1, OUT))


@jax.jit
def kernel(x, edge_index, W_proj, b_proj, W_msg, b_msg, W_next, b_next,
           gamma, beta, W_out, b_out):
    h, a, b = _tc_pre(x, W_proj, b_proj, W_msg, b_msg)
    src = edge_index[0]
    dst = edge_index[1]
    partials = _sc_edge(a, b, src, dst)
    counts = _tc_hist(dst)
    return _tc_post(h, partials, counts, W_next, b_next,
                    gamma, beta, W_out, b_out)
